# baseline (speedup 1.0000x reference)
import functools

import jax
import jax.numpy as jnp
from jax import lax
from jax.experimental import pallas as pl
from jax.experimental.pallas import tpu as pltpu
from jax.experimental.pallas import tpu_sc as plsc

T, N, E, H = 3, 10000, 320000, 128
NC, NS = 2, 16
NPAD = 10240
RPS = NPAD // NS
K = 128
EP = 327680
RT = EP // K
RB = 2000
NBLK = N // RB

_vmesh = plsc.VectorSubcoreMesh(core_axis_name="c", subcore_axis_name="s")


def _deg_body(dst_hbm, out_hbm, acc0, acc1, acc2, zbuf, ones, dstv):
    c = lax.axis_index("c")
    s = lax.axis_index("s")
    accs = [acc0, acc1, acc2]

    zero16 = jnp.zeros((16,), jnp.float32)
    one16 = jnp.ones((16,), jnp.float32)

    def fill_z(i, _):
        zbuf[i, :] = zero16
        return 0

    lax.fori_loop(0, 128, fill_z, 0)

    def fill_o(i, _):
        ones[i, :] = one16
        return 0

    lax.fori_loop(0, K, fill_o, 0)

    for t in range(T):
        for r in range(5):
            pltpu.sync_copy(zbuf,
                            accs[t].at[pl.ds(s * 640 + r * 128, 128), :])
    plsc.subcore_barrier()

    per_sub = EP // (NC * NS)
    for t in range(T):
        acc = accs[t]
        tbase = t * EP + c * (EP // NC) + s * per_sub

        def body(g, _):
            pltpu.sync_copy(dst_hbm.at[pl.ds(tbase + g * K, K)], dstv)
            pltpu.sync_copy(ones, acc.at[dstv], add=True)
            return 0

        lax.fori_loop(0, per_sub // K, body, 0)
    plsc.subcore_barrier()

    for t in range(T):
        pltpu.sync_copy(accs[t].at[pl.ds(s * 640, 640), :],
                        out_hbm.at[c * T + t, pl.ds(s * 640, 640), :])


def _zero_acc(acc, zbuf, s):
    zero16 = jnp.zeros((16,), jnp.float32)

    def fill_z(k, _):
        i = k // (H // 16)
        j = k % (H // 16)
        zbuf[i, pl.ds(j * 16, 16)] = zero16
        return 0

    lax.fori_loop(0, 32 * (H // 16), fill_z, 0)
    for r in range(RPS // 32):
        pltpu.sync_copy(zbuf, acc.at[pl.ds(s * RPS + r * 32, 32), :])


def _edge_loop(table_hbm, src_hbm, dst_hbm, acc, srcv0, srcv1, dstv0, dstv1,
               rows0, rows1, gsem0, gsem1, base, n_chunks):
    srcv = [srcv0, srcv1]
    dstv = [dstv0, dstv1]
    rows = [rows0, rows1]
    gsems = [gsem0, gsem1]

    def body(j, _):
        e0 = base + j * (2 * K)
        pltpu.sync_copy(src_hbm.at[pl.ds(e0, K)], srcv[0])
        pltpu.sync_copy(dst_hbm.at[pl.ds(e0, K)], dstv[0])
        pltpu.async_copy(table_hbm.at[srcv[0]], rows[0], gsems[0]).wait()
        pltpu.sync_copy(rows[0], acc.at[dstv[0]], add=True)
        pltpu.sync_copy(src_hbm.at[pl.ds(e0 + K, K)], srcv[1])
        pltpu.sync_copy(dst_hbm.at[pl.ds(e0 + K, K)], dstv[1])
        pltpu.async_copy(table_hbm.at[srcv[1]], rows[1], gsems[1]).wait()
        pltpu.sync_copy(rows[1], acc.at[dstv[1]], add=True)
        return 0

    lax.fori_loop(0, n_chunks // 2, body, 0)


def _prop1_body(u_hbm, src_hbm, dst_hbm, out_hbm, acc, zbuf, srcv0, srcv1,
                dstv0, dstv1, r0, r1, gsem0, gsem1):
    c = lax.axis_index("c")
    s = lax.axis_index("s")
    _zero_acc(acc, zbuf, s)
    plsc.subcore_barrier()
    per_sub = EP // (NC * NS)
    base = c * (EP // NC) + s * per_sub
    _edge_loop(u_hbm, src_hbm, dst_hbm, acc, srcv0, srcv1, dstv0, dstv1,
               r0, r1, gsem0, gsem1, base, per_sub // K)
    plsc.subcore_barrier()
    pltpu.sync_copy(acc.at[pl.ds(s * RPS, RPS), :],
                    out_hbm.at[c, pl.ds(s * RPS, RPS), :])


def _prop2_body(ua_hbm, ub_hbm, src_hbm, dst_hbm, out_hbm, acc, zbuf, srcv0,
                srcv1, dstv0, dstv1, r0, r1, gsem0, gsem1):
    c = lax.axis_index("c")
    s = lax.axis_index("s")
    _zero_acc(acc, zbuf, s)
    plsc.subcore_barrier()
    per_sub = EP // NS
    base = s * per_sub

    @pl.when(c == 0)
    def _():
        _edge_loop(ua_hbm, src_hbm, dst_hbm, acc, srcv0, srcv1, dstv0, dstv1,
                   r0, r1, gsem0, gsem1, base, per_sub // K)

    @pl.when(c == 1)
    def _():
        _edge_loop(ub_hbm, src_hbm, dst_hbm, acc, srcv0, srcv1, dstv0, dstv1,
                   r0, r1, gsem0, gsem1, base, per_sub // K)

    plsc.subcore_barrier()
    pltpu.sync_copy(acc.at[pl.ds(s * RPS, RPS), :],
                    out_hbm.at[c, pl.ds(s * RPS, RPS), :])


_PROP_SCRATCH = [
    pltpu.VMEM_SHARED((NPAD, H), jnp.float32),
    pltpu.VMEM((32, H), jnp.float32),
    pltpu.VMEM((K,), jnp.int32),
    pltpu.VMEM((K,), jnp.int32),
    pltpu.VMEM((K,), jnp.int32),
    pltpu.VMEM((K,), jnp.int32),
    pltpu.VMEM((K, H), jnp.float32),
    pltpu.VMEM((K, H), jnp.float32),
    pltpu.SemaphoreType.DMA,
    pltpu.SemaphoreType.DMA,
]

_deg_kernel = functools.partial(
    pl.kernel,
    out_type=jax.ShapeDtypeStruct((NC * T, NPAD, 16), jnp.float32),
    mesh=_vmesh,
    scratch_types=[
        pltpu.VMEM_SHARED((NPAD, 16), jnp.float32),
        pltpu.VMEM_SHARED((NPAD, 16), jnp.float32),
        pltpu.VMEM_SHARED((NPAD, 16), jnp.float32),
        pltpu.VMEM((128, 16), jnp.float32),
        pltpu.VMEM((K, 16), jnp.float32),
        pltpu.VMEM((K,), jnp.int32),
    ],
)(_deg_body)

_prop1_kernel = functools.partial(
    pl.kernel,
    out_type=jax.ShapeDtypeStruct((NC, NPAD, H), jnp.float32),
    mesh=_vmesh,
    scratch_types=_PROP_SCRATCH,
)(_prop1_body)

_prop2_kernel = functools.partial(
    pl.kernel,
    out_type=jax.ShapeDtypeStruct((NC, NPAD, H), jnp.float32),
    mesh=_vmesh,
    scratch_types=_PROP_SCRATCH,
)(_prop2_body)


def _scale_body(dp_ref, xs_ref, ux_ref, db_ref):
    dp = dp_ref[...]
    deg = dp[0, 0, :, 0:1] + dp[1, 0, :, 0:1] + 1.0
    dinv = lax.rsqrt(deg)
    ux_ref[0] = xs_ref[0] * dinv
    db_ref[0] = jnp.broadcast_to(dinv, (RB, H))


def _tc_scale(dp, xs):
    return pl.pallas_call(
        _scale_body,
        grid=(T, NBLK),
        in_specs=[
            pl.BlockSpec((2, 1, RB, 16), lambda t, b: (0, t, b, 0)),
            pl.BlockSpec((1, RB, H), lambda t, b: (t, b, 0)),
        ],
        out_specs=[
            pl.BlockSpec((1, RB, H), lambda t, b: (t, b, 0)),
            pl.BlockSpec((1, RB, H), lambda t, b: (t, b, 0)),
        ],
        out_shape=[
            jax.ShapeDtypeStruct((T, N, H), jnp.float32),
            jax.ShapeDtypeStruct((T, N, H), jnp.float32),
        ],
    )(dp, xs)


def _gates0_body(pp_ref, ux_ref, db0_ref, db1_ref, wz_ref, wh_ref, bz_ref,
                 bh_ref, h_ref, uh_ref):
    p = (pp_ref[0] + pp_ref[1] + ux_ref[...]) * db0_ref[...]
    z = jax.nn.sigmoid(
        jnp.dot(p, wz_ref[...], preferred_element_type=jnp.float32)
        + bz_ref[...])
    hh = jnp.tanh(
        jnp.dot(p, wh_ref[...], preferred_element_type=jnp.float32)
        + bh_ref[...])
    h1 = (1.0 - z) * hh
    h_ref[...] = h1
    uh_ref[...] = h1 * db1_ref[...]


def _tc_gates0(pp, ux, db0, db1, wz, wh, bz, bh):
    row = pl.BlockSpec((RB, H), lambda b: (b, 0))
    full = pl.BlockSpec((H, H), lambda b: (0, 0))
    bias = pl.BlockSpec((1, H), lambda b: (0, 0))
    return pl.pallas_call(
        _gates0_body,
        grid=(NBLK,),
        in_specs=[pl.BlockSpec((2, RB, H), lambda b: (0, b, 0)),
                  row, row, row, full, full, bias, bias],
        out_specs=[row, row],
        out_shape=[jax.ShapeDtypeStruct((N, H), jnp.float32),
                   jax.ShapeDtypeStruct((N, H), jnp.float32)],
    )(pp, ux, db0, db1, wz, wh, bz, bh)


def _gates_body(pf_ref, qf_ref, ux_ref, uh_ref, db_ref, h_ref, wxz_ref,
                whz_ref, wxr_ref, whr_ref, wxh_ref, bz_ref, br_ref, bxh_ref,
                z_ref, xh_ref, ug_ref):
    db = db_ref[...]
    p = (pf_ref[...] + ux_ref[...]) * db
    q = (qf_ref[...] + uh_ref[...]) * db
    dot = lambda a, w: jnp.dot(a, w, preferred_element_type=jnp.float32)
    z = jax.nn.sigmoid(dot(p, wxz_ref[...]) + dot(q, whz_ref[...])
                       + bz_ref[...])
    r = jax.nn.sigmoid(dot(p, wxr_ref[...]) + dot(q, whr_ref[...])
                       + br_ref[...])
    z_ref[...] = z
    xh_ref[...] = dot(p, wxh_ref[...]) + bxh_ref[...]
    ug_ref[...] = r * h_ref[...] * db


def _tc_gates(pf, qf, ux, uh, db, h, wxz, whz, wxr, whr, wxh, bz, br, bxh):
    row = pl.BlockSpec((RB, H), lambda b: (b, 0))
    full = pl.BlockSpec((H, H), lambda b: (0, 0))
    bias = pl.BlockSpec((1, H), lambda b: (0, 0))
    return pl.pallas_call(
        _gates_body,
        grid=(NBLK,),
        in_specs=[row, row, row, row, row, row,
                  full, full, full, full, full, bias, bias, bias],
        out_specs=[row, row, row],
        out_shape=[jax.ShapeDtypeStruct((N, H), jnp.float32),
                   jax.ShapeDtypeStruct((N, H), jnp.float32),
                   jax.ShapeDtypeStruct((N, H), jnp.float32)],
    )(pf, qf, ux, uh, db, h, wxz, whz, wxr, whr, wxh, bz, br, bxh)


def _final_body(sp_ref, ug_ref, db_ref, xh_ref, z_ref, h0_ref, whh_ref,
                bhh_ref, dbn_ref, h_ref, uhn_ref):
    sv = (sp_ref[0] + sp_ref[1] + ug_ref[...]) * db_ref[...]
    hh = jnp.tanh(
        xh_ref[...]
        + jnp.dot(sv, whh_ref[...], preferred_element_type=jnp.float32)
        + bhh_ref[...])
    z = z_ref[...]
    hn = z * h0_ref[...] + (1.0 - z) * hh
    h_ref[...] = hn
    uhn_ref[...] = hn * dbn_ref[...]


def _tc_final(sp, ug, db, xh, z, h0, whh, bhh, dbn):
    row = pl.BlockSpec((RB, H), lambda b: (b, 0))
    full = pl.BlockSpec((H, H), lambda b: (0, 0))
    bias = pl.BlockSpec((1, H), lambda b: (0, 0))
    return pl.pallas_call(
        _final_body,
        grid=(NBLK,),
        in_specs=[pl.BlockSpec((2, RB, H), lambda b: (0, b, 0)),
                  row, row, row, row, bias, full, bias, row],
        out_specs=[row, row],
        out_shape=[jax.ShapeDtypeStruct((N, H), jnp.float32),
                   jax.ShapeDtypeStruct((N, H), jnp.float32)],
    )(sp, ug, db, xh, z, h0, whh, bhh, dbn)


def kernel(xs, eis, W_xz, b_xz, W_hz, b_hz, W_xr, b_xr, W_hr, b_hr,
           W_xh, b_xh, W_hh, b_hh):
    pad_src = jnp.zeros((T, EP - E), jnp.int32)
    pad_dst = jnp.full((T, EP - E), NPAD - 1, jnp.int32)
    srcs = jnp.concatenate([eis[:, 0, :], pad_src], axis=1)
    dsts = jnp.concatenate([eis[:, 1, :], pad_dst], axis=1)

    deg_parts = _deg_kernel(dsts.reshape(-1))
    dp = deg_parts.reshape(NC, T, NPAD, 16)[:, :, :N, :]
    ux3, db3 = _tc_scale(dp, xs)

    bz = (b_xz + b_hz).reshape(1, H)
    br = (b_xr + b_hr).reshape(1, H)
    bh0 = (b_xh + b_hh).reshape(1, H)
    bxh = b_xh.reshape(1, H)
    bhh = b_hh.reshape(1, H)

    pp = _prop1_kernel(ux3[0], srcs[0], dsts[0])[:, :N, :]
    h, uh = _tc_gates0(pp, ux3[0], db3[0], db3[1], W_xz, W_xh, bz, bh0)
    hs = [h]
    for t in (1, 2):
        pq = _prop2_kernel(ux3[t], uh, srcs[t], dsts[t])[:, :N, :]
        z, xh, ug = _tc_gates(pq[0], pq[1], ux3[t], uh, db3[t], h,
                              W_xz, W_hz, W_xr, W_hr, W_xh, bz, br, bxh)
        sp = _prop1_kernel(ug, srcs[t], dsts[t])[:, :N, :]
        dbn = db3[t + 1] if t + 1 < T else db3[t]
        h, uh = _tc_final(sp, ug, db3[t], xh, z, h[0:1], W_hh, bhh, dbn)
        hs.append(h)
    return jnp.stack(hs)


# speedup vs baseline: 1.0530x; 1.0530x over previous
import functools

import jax
import jax.numpy as jnp
from jax import lax
from jax.experimental import pallas as pl
from jax.experimental.pallas import tpu as pltpu
from jax.experimental.pallas import tpu_sc as plsc

T, N, E, H = 3, 10000, 320000, 128
NC, NS = 2, 16
NPAD = 10240
RPS = NPAD // NS
K = 80
EP = 327680
RT = EP // K
RB = 2000
NBLK = N // RB

_vmesh = plsc.VectorSubcoreMesh(core_axis_name="c", subcore_axis_name="s")


def _deg_body(dst_hbm, out_hbm, acc0, acc1, acc2, zbuf, ones, dstv):
    c = lax.axis_index("c")
    s = lax.axis_index("s")
    accs = [acc0, acc1, acc2]

    zero16 = jnp.zeros((16,), jnp.float32)
    one16 = jnp.ones((16,), jnp.float32)

    def fill_z(i, _):
        zbuf[i, :] = zero16
        return 0

    lax.fori_loop(0, 128, fill_z, 0)

    def fill_o(i, _):
        ones[i, :] = one16
        return 0

    lax.fori_loop(0, K, fill_o, 0)

    for t in range(T):
        for r in range(5):
            pltpu.sync_copy(zbuf,
                            accs[t].at[pl.ds(s * 640 + r * 128, 128), :])
    plsc.subcore_barrier()

    per_sub = EP // (NC * NS)
    for t in range(T):
        acc = accs[t]
        tbase = t * EP + c * (EP // NC) + s * per_sub

        def body(g, _):
            pltpu.sync_copy(dst_hbm.at[pl.ds(tbase + g * K, K)], dstv)
            pltpu.sync_copy(ones, acc.at[dstv], add=True)
            return 0

        lax.fori_loop(0, per_sub // K, body, 0)
    plsc.subcore_barrier()

    for t in range(T):
        pltpu.sync_copy(accs[t].at[pl.ds(s * 640, 640), :],
                        out_hbm.at[c * T + t, pl.ds(s * 640, 640), :])


def _zero_acc(acc, zbuf, s):
    zero16 = jnp.zeros((16,), jnp.float32)

    def fill_z(k, _):
        i = k // (H // 16)
        j = k % (H // 16)
        zbuf[i, pl.ds(j * 16, 16)] = zero16
        return 0

    lax.fori_loop(0, 32 * (H // 16), fill_z, 0)
    for r in range(RPS // 32):
        pltpu.sync_copy(zbuf, acc.at[pl.ds(s * RPS + r * 32, 32), :])


def _edge_loop(table_hbm, src_hbm, dst_hbm, acc, srcv0, srcv1, dstv0, dstv1,
               rows0, rows1, gsem0, gsem1, base, n_chunks):
    srcv = [srcv0, srcv1]
    dstv = [dstv0, dstv1]
    rows = [rows0, rows1]
    gsems = [gsem0, gsem1]

    def body(j, _):
        e0 = base + j * (2 * K)
        gcp = [None, None]
        pltpu.sync_copy(src_hbm.at[pl.ds(e0, K)], srcv[0])
        pltpu.sync_copy(dst_hbm.at[pl.ds(e0, K)], dstv[0])
        gcp[0] = pltpu.async_copy(table_hbm.at[srcv[0]], rows[0], gsems[0])
        pltpu.sync_copy(src_hbm.at[pl.ds(e0 + K, K)], srcv[1])
        pltpu.sync_copy(dst_hbm.at[pl.ds(e0 + K, K)], dstv[1])
        gcp[1] = pltpu.async_copy(table_hbm.at[srcv[1]], rows[1], gsems[1])
        gcp[0].wait()
        pltpu.sync_copy(rows[0], acc.at[dstv[0]], add=True)
        gcp[1].wait()
        pltpu.sync_copy(rows[1], acc.at[dstv[1]], add=True)
        return 0

    lax.fori_loop(0, n_chunks // 2, body, 0)


def _prop1_body(u_hbm, src_hbm, dst_hbm, out_hbm, acc, zbuf, srcv0, srcv1,
                dstv0, dstv1, r0, r1, gsem0, gsem1):
    c = lax.axis_index("c")
    s = lax.axis_index("s")
    _zero_acc(acc, zbuf, s)
    plsc.subcore_barrier()
    per_sub = EP // (NC * NS)
    base = c * (EP // NC) + s * per_sub
    _edge_loop(u_hbm, src_hbm, dst_hbm, acc, srcv0, srcv1, dstv0, dstv1,
               r0, r1, gsem0, gsem1, base, per_sub // K)
    plsc.subcore_barrier()
    pltpu.sync_copy(acc.at[pl.ds(s * RPS, RPS), :],
                    out_hbm.at[c, pl.ds(s * RPS, RPS), :])


def _prop2_body(ua_hbm, ub_hbm, src_hbm, dst_hbm, out_hbm, acc, zbuf, srcv0,
                srcv1, dstv0, dstv1, r0, r1, gsem0, gsem1):
    c = lax.axis_index("c")
    s = lax.axis_index("s")
    _zero_acc(acc, zbuf, s)
    plsc.subcore_barrier()
    per_sub = EP // NS
    base = s * per_sub

    @pl.when(c == 0)
    def _():
        _edge_loop(ua_hbm, src_hbm, dst_hbm, acc, srcv0, srcv1, dstv0, dstv1,
                   r0, r1, gsem0, gsem1, base, per_sub // K)

    @pl.when(c == 1)
    def _():
        _edge_loop(ub_hbm, src_hbm, dst_hbm, acc, srcv0, srcv1, dstv0, dstv1,
                   r0, r1, gsem0, gsem1, base, per_sub // K)

    plsc.subcore_barrier()
    pltpu.sync_copy(acc.at[pl.ds(s * RPS, RPS), :],
                    out_hbm.at[c, pl.ds(s * RPS, RPS), :])


_PROP_SCRATCH = [
    pltpu.VMEM_SHARED((NPAD, H), jnp.float32),
    pltpu.VMEM((32, H), jnp.float32),
    pltpu.VMEM((K,), jnp.int32),
    pltpu.VMEM((K,), jnp.int32),
    pltpu.VMEM((K,), jnp.int32),
    pltpu.VMEM((K,), jnp.int32),
    pltpu.VMEM((K, H), jnp.float32),
    pltpu.VMEM((K, H), jnp.float32),
    pltpu.SemaphoreType.DMA,
    pltpu.SemaphoreType.DMA,
]

_deg_kernel = functools.partial(
    pl.kernel,
    out_type=jax.ShapeDtypeStruct((NC * T, NPAD, 16), jnp.float32),
    mesh=_vmesh,
    scratch_types=[
        pltpu.VMEM_SHARED((NPAD, 16), jnp.float32),
        pltpu.VMEM_SHARED((NPAD, 16), jnp.float32),
        pltpu.VMEM_SHARED((NPAD, 16), jnp.float32),
        pltpu.VMEM((128, 16), jnp.float32),
        pltpu.VMEM((K, 16), jnp.float32),
        pltpu.VMEM((K,), jnp.int32),
    ],
)(_deg_body)

_prop1_kernel = functools.partial(
    pl.kernel,
    out_type=jax.ShapeDtypeStruct((NC, NPAD, H), jnp.float32),
    mesh=_vmesh,
    scratch_types=_PROP_SCRATCH,
)(_prop1_body)

_prop2_kernel = functools.partial(
    pl.kernel,
    out_type=jax.ShapeDtypeStruct((NC, NPAD, H), jnp.float32),
    mesh=_vmesh,
    scratch_types=_PROP_SCRATCH,
)(_prop2_body)


def _scale_body(dp_ref, xs_ref, ux_ref, db_ref):
    dp = dp_ref[...]
    deg = dp[0, 0, :, 0:1] + dp[1, 0, :, 0:1] + 1.0
    dinv = lax.rsqrt(deg)
    ux_ref[0] = xs_ref[0] * dinv
    db_ref[0] = jnp.broadcast_to(dinv, (RB, H))


def _tc_scale(dp, xs):
    return pl.pallas_call(
        _scale_body,
        grid=(T, NBLK),
        in_specs=[
            pl.BlockSpec((2, 1, RB, 16), lambda t, b: (0, t, b, 0)),
            pl.BlockSpec((1, RB, H), lambda t, b: (t, b, 0)),
        ],
        out_specs=[
            pl.BlockSpec((1, RB, H), lambda t, b: (t, b, 0)),
            pl.BlockSpec((1, RB, H), lambda t, b: (t, b, 0)),
        ],
        out_shape=[
            jax.ShapeDtypeStruct((T, N, H), jnp.float32),
            jax.ShapeDtypeStruct((T, N, H), jnp.float32),
        ],
    )(dp, xs)


def _gates0_body(pp_ref, ux_ref, db0_ref, db1_ref, wz_ref, wh_ref, bz_ref,
                 bh_ref, h_ref, uh_ref):
    p = (pp_ref[0] + pp_ref[1] + ux_ref[...]) * db0_ref[...]
    z = jax.nn.sigmoid(
        jnp.dot(p, wz_ref[...], preferred_element_type=jnp.float32)
        + bz_ref[...])
    hh = jnp.tanh(
        jnp.dot(p, wh_ref[...], preferred_element_type=jnp.float32)
        + bh_ref[...])
    h1 = (1.0 - z) * hh
    h_ref[...] = h1
    uh_ref[...] = h1 * db1_ref[...]


def _tc_gates0(pp, ux, db0, db1, wz, wh, bz, bh):
    row = pl.BlockSpec((RB, H), lambda b: (b, 0))
    full = pl.BlockSpec((H, H), lambda b: (0, 0))
    bias = pl.BlockSpec((1, H), lambda b: (0, 0))
    return pl.pallas_call(
        _gates0_body,
        grid=(NBLK,),
        in_specs=[pl.BlockSpec((2, RB, H), lambda b: (0, b, 0)),
                  row, row, row, full, full, bias, bias],
        out_specs=[row, row],
        out_shape=[jax.ShapeDtypeStruct((N, H), jnp.float32),
                   jax.ShapeDtypeStruct((N, H), jnp.float32)],
    )(pp, ux, db0, db1, wz, wh, bz, bh)


def _gates_body(pf_ref, qf_ref, ux_ref, uh_ref, db_ref, h_ref, wxz_ref,
                whz_ref, wxr_ref, whr_ref, wxh_ref, bz_ref, br_ref, bxh_ref,
                z_ref, xh_ref, ug_ref):
    db = db_ref[...]
    p = (pf_ref[...] + ux_ref[...]) * db
    q = (qf_ref[...] + uh_ref[...]) * db
    dot = lambda a, w: jnp.dot(a, w, preferred_element_type=jnp.float32)
    z = jax.nn.sigmoid(dot(p, wxz_ref[...]) + dot(q, whz_ref[...])
                       + bz_ref[...])
    r = jax.nn.sigmoid(dot(p, wxr_ref[...]) + dot(q, whr_ref[...])
                       + br_ref[...])
    z_ref[...] = z
    xh_ref[...] = dot(p, wxh_ref[...]) + bxh_ref[...]
    ug_ref[...] = r * h_ref[...] * db


def _tc_gates(pf, qf, ux, uh, db, h, wxz, whz, wxr, whr, wxh, bz, br, bxh):
    row = pl.BlockSpec((RB, H), lambda b: (b, 0))
    full = pl.BlockSpec((H, H), lambda b: (0, 0))
    bias = pl.BlockSpec((1, H), lambda b: (0, 0))
    return pl.pallas_call(
        _gates_body,
        grid=(NBLK,),
        in_specs=[row, row, row, row, row, row,
                  full, full, full, full, full, bias, bias, bias],
        out_specs=[row, row, row],
        out_shape=[jax.ShapeDtypeStruct((N, H), jnp.float32),
                   jax.ShapeDtypeStruct((N, H), jnp.float32),
                   jax.ShapeDtypeStruct((N, H), jnp.float32)],
    )(pf, qf, ux, uh, db, h, wxz, whz, wxr, whr, wxh, bz, br, bxh)


def _final_body(sp_ref, ug_ref, db_ref, xh_ref, z_ref, h0_ref, whh_ref,
                bhh_ref, dbn_ref, h_ref, uhn_ref):
    sv = (sp_ref[0] + sp_ref[1] + ug_ref[...]) * db_ref[...]
    hh = jnp.tanh(
        xh_ref[...]
        + jnp.dot(sv, whh_ref[...], preferred_element_type=jnp.float32)
        + bhh_ref[...])
    z = z_ref[...]
    hn = z * h0_ref[...] + (1.0 - z) * hh
    h_ref[...] = hn
    uhn_ref[...] = hn * dbn_ref[...]


def _tc_final(sp, ug, db, xh, z, h0, whh, bhh, dbn):
    row = pl.BlockSpec((RB, H), lambda b: (b, 0))
    full = pl.BlockSpec((H, H), lambda b: (0, 0))
    bias = pl.BlockSpec((1, H), lambda b: (0, 0))
    return pl.pallas_call(
        _final_body,
        grid=(NBLK,),
        in_specs=[pl.BlockSpec((2, RB, H), lambda b: (0, b, 0)),
                  row, row, row, row, bias, full, bias, row],
        out_specs=[row, row],
        out_shape=[jax.ShapeDtypeStruct((N, H), jnp.float32),
                   jax.ShapeDtypeStruct((N, H), jnp.float32)],
    )(sp, ug, db, xh, z, h0, whh, bhh, dbn)


def kernel(xs, eis, W_xz, b_xz, W_hz, b_hz, W_xr, b_xr, W_hr, b_hr,
           W_xh, b_xh, W_hh, b_hh):
    pad_src = jnp.zeros((T, EP - E), jnp.int32)
    pad_dst = jnp.full((T, EP - E), NPAD - 1, jnp.int32)
    srcs = jnp.concatenate([eis[:, 0, :], pad_src], axis=1)
    dsts = jnp.concatenate([eis[:, 1, :], pad_dst], axis=1)

    deg_parts = _deg_kernel(dsts.reshape(-1))
    dp = deg_parts.reshape(NC, T, NPAD, 16)[:, :, :N, :]
    ux3, db3 = _tc_scale(dp, xs)

    bz = (b_xz + b_hz).reshape(1, H)
    br = (b_xr + b_hr).reshape(1, H)
    bh0 = (b_xh + b_hh).reshape(1, H)
    bxh = b_xh.reshape(1, H)
    bhh = b_hh.reshape(1, H)

    pp = _prop1_kernel(ux3[0], srcs[0], dsts[0])[:, :N, :]
    h, uh = _tc_gates0(pp, ux3[0], db3[0], db3[1], W_xz, W_xh, bz, bh0)
    hs = [h]
    for t in (1, 2):
        pq = _prop2_kernel(ux3[t], uh, srcs[t], dsts[t])[:, :N, :]
        z, xh, ug = _tc_gates(pq[0], pq[1], ux3[t], uh, db3[t], h,
                              W_xz, W_hz, W_xr, W_hr, W_xh, bz, br, bxh)
        sp = _prop1_kernel(ug, srcs[t], dsts[t])[:, :N, :]
        dbn = db3[t + 1] if t + 1 < T else db3[t]
        h, uh = _tc_final(sp, ug, db3[t], xh, z, h[0:1], W_hh, bhh, dbn)
        hs.append(h)
    return jnp.stack(hs)


# speedup vs baseline: 1.3856x; 1.3159x over previous
import functools

import jax
import jax.numpy as jnp
from jax import lax
from jax.experimental import pallas as pl
from jax.experimental.pallas import tpu as pltpu
from jax.experimental.pallas import tpu_sc as plsc

T, N, E, H = 3, 10000, 320000, 128
NC, NS = 2, 16
NPAD = 10240
RPS = NPAD // NS
K = 80
RB = 2000
NBLK = N // RB

_vmesh = plsc.VectorSubcoreMesh(core_axis_name="c", subcore_axis_name="s")


def _deg_body(dst_hbm, out_hbm, acc0, acc1, acc2, zbuf, ones, dstv):
    c = lax.axis_index("c")
    s = lax.axis_index("s")
    accs = [acc0, acc1, acc2]

    zero16 = jnp.zeros((16,), jnp.float32)
    one16 = jnp.ones((16,), jnp.float32)

    def fill_z(i, _):
        zbuf[i, :] = zero16
        return 0

    lax.fori_loop(0, 640, fill_z, 0)

    def fill_o(i, _):
        ones[i, :] = one16
        return 0

    lax.fori_loop(0, K, fill_o, 0)

    for t in range(T):
        pltpu.sync_copy(zbuf, accs[t].at[pl.ds(s * 640, 640), :])
    plsc.subcore_barrier()

    per_sub = E // (NC * NS)
    base = c * (E // NC) + s * per_sub
    for t in range(T):
        acc = accs[t]
        tbase = t * E + base

        def body(g, _):
            pltpu.sync_copy(dst_hbm.at[pl.ds(tbase + g * K, K)], dstv)
            pltpu.sync_copy(ones, acc.at[dstv], add=True)
            return 0

        lax.fori_loop(0, per_sub // K, body, 0)
    plsc.subcore_barrier()

    for t in range(T):
        pltpu.sync_copy(accs[t].at[pl.ds(s * 640, 640), :],
                        out_hbm.at[c * T + t, pl.ds(s * 640, 640), :])


def _zero_acc(acc, zbuf, s):
    zero16 = jnp.zeros((16,), jnp.float32)

    def fill_z(k, _):
        i = k // (H // 16)
        j = k % (H // 16)
        zbuf[i, pl.ds(j * 16, 16)] = zero16
        return 0

    lax.fori_loop(0, 128 * (H // 16), fill_z, 0)
    for r in range(RPS // 128):
        pltpu.sync_copy(zbuf, acc.at[pl.ds(s * RPS + r * 128, 128), :])


def _edge_loop(table_hbm, src_hbm, dst_hbm, acc, srcv, dstv, rows, sem,
               base, n_chunks):
    def body(g, _):
        off = base + g * K
        pltpu.sync_copy(src_hbm.at[pl.ds(off, K)], srcv)
        pltpu.sync_copy(dst_hbm.at[pl.ds(off, K)], dstv)
        pltpu.async_copy(table_hbm.at[srcv], rows, sem).wait()
        pltpu.sync_copy(rows, acc.at[dstv], add=True)
        return 0

    lax.fori_loop(0, n_chunks, body, 0)


def _prop1_body(u_hbm, src_hbm, dst_hbm, out_hbm, acc, zbuf, srcv, dstv, rows,
                sem):
    c = lax.axis_index("c")
    s = lax.axis_index("s")
    _zero_acc(acc, zbuf, s)
    plsc.subcore_barrier()
    per_sub = E // (NC * NS)
    base = c * (E // NC) + s * per_sub
    _edge_loop(u_hbm, src_hbm, dst_hbm, acc, srcv, dstv, rows, sem,
               base, per_sub // K)
    plsc.subcore_barrier()
    pltpu.sync_copy(acc.at[pl.ds(s * RPS, RPS), :],
                    out_hbm.at[c, pl.ds(s * RPS, RPS), :])


def _prop2_body(ua_hbm, ub_hbm, src_hbm, dst_hbm, out_hbm, acc, zbuf, srcv,
                dstv, rows, sem):
    c = lax.axis_index("c")
    s = lax.axis_index("s")
    _zero_acc(acc, zbuf, s)
    plsc.subcore_barrier()
    per_sub = E // NS
    base = s * per_sub

    @pl.when(c == 0)
    def _():
        _edge_loop(ua_hbm, src_hbm, dst_hbm, acc, srcv, dstv, rows, sem,
                   base, per_sub // K)

    @pl.when(c == 1)
    def _():
        _edge_loop(ub_hbm, src_hbm, dst_hbm, acc, srcv, dstv, rows, sem,
                   base, per_sub // K)

    plsc.subcore_barrier()
    pltpu.sync_copy(acc.at[pl.ds(s * RPS, RPS), :],
                    out_hbm.at[c, pl.ds(s * RPS, RPS), :])


_PROP_SCRATCH = [
    pltpu.VMEM_SHARED((NPAD, H), jnp.float32),
    pltpu.VMEM((128, H), jnp.float32),
    pltpu.VMEM((K,), jnp.int32),
    pltpu.VMEM((K,), jnp.int32),
    pltpu.VMEM((K, H), jnp.float32),
    pltpu.SemaphoreType.DMA,
]

_deg_kernel = functools.partial(
    pl.kernel,
    out_type=jax.ShapeDtypeStruct((NC * T, NPAD, 16), jnp.float32),
    mesh=_vmesh,
    scratch_types=[
        pltpu.VMEM_SHARED((NPAD, 16), jnp.float32),
        pltpu.VMEM_SHARED((NPAD, 16), jnp.float32),
        pltpu.VMEM_SHARED((NPAD, 16), jnp.float32),
        pltpu.VMEM((640, 16), jnp.float32),
        pltpu.VMEM((K, 16), jnp.float32),
        pltpu.VMEM((K,), jnp.int32),
    ],
)(_deg_body)

_prop1_kernel = functools.partial(
    pl.kernel,
    out_type=jax.ShapeDtypeStruct((NC, NPAD, H), jnp.float32),
    mesh=_vmesh,
    scratch_types=_PROP_SCRATCH,
)(_prop1_body)

_prop2_kernel = functools.partial(
    pl.kernel,
    out_type=jax.ShapeDtypeStruct((NC, NPAD, H), jnp.float32),
    mesh=_vmesh,
    scratch_types=_PROP_SCRATCH,
)(_prop2_body)


def _scale_body(dp_ref, xs_ref, ux_ref, db_ref):
    dp = dp_ref[...]
    deg = dp[0, 0, :, 0:1] + dp[1, 0, :, 0:1] + 1.0
    dinv = lax.rsqrt(deg)
    ux_ref[0] = xs_ref[0] * dinv
    db_ref[0] = jnp.broadcast_to(dinv, (RB, H))


def _tc_scale(dp, xs):
    return pl.pallas_call(
        _scale_body,
        grid=(T, NBLK),
        in_specs=[
            pl.BlockSpec((2, 1, RB, 16), lambda t, b: (0, t, b, 0)),
            pl.BlockSpec((1, RB, H), lambda t, b: (t, b, 0)),
        ],
        out_specs=[
            pl.BlockSpec((1, RB, H), lambda t, b: (t, b, 0)),
            pl.BlockSpec((1, RB, H), lambda t, b: (t, b, 0)),
        ],
        out_shape=[
            jax.ShapeDtypeStruct((T, N, H), jnp.float32),
            jax.ShapeDtypeStruct((T, N, H), jnp.float32),
        ],
    )(dp, xs)


def _gates0_body(pp_ref, ux_ref, db0_ref, db1_ref, wz_ref, wh_ref, bz_ref,
                 bh_ref, h_ref, uh_ref):
    p = (pp_ref[0] + pp_ref[1] + ux_ref[...]) * db0_ref[...]
    z = jax.nn.sigmoid(
        jnp.dot(p, wz_ref[...], preferred_element_type=jnp.float32)
        + bz_ref[...])
    hh = jnp.tanh(
        jnp.dot(p, wh_ref[...], preferred_element_type=jnp.float32)
        + bh_ref[...])
    h1 = (1.0 - z) * hh
    h_ref[...] = h1
    uh_ref[...] = h1 * db1_ref[...]


def _tc_gates0(pp, ux, db0, db1, wz, wh, bz, bh):
    row = pl.BlockSpec((RB, H), lambda b: (b, 0))
    full = pl.BlockSpec((H, H), lambda b: (0, 0))
    bias = pl.BlockSpec((1, H), lambda b: (0, 0))
    return pl.pallas_call(
        _gates0_body,
        grid=(NBLK,),
        in_specs=[pl.BlockSpec((2, RB, H), lambda b: (0, b, 0)),
                  row, row, row, full, full, bias, bias],
        out_specs=[row, row],
        out_shape=[jax.ShapeDtypeStruct((N, H), jnp.float32),
                   jax.ShapeDtypeStruct((N, H), jnp.float32)],
    )(pp, ux, db0, db1, wz, wh, bz, bh)


def _gates_body(pf_ref, qf_ref, ux_ref, uh_ref, db_ref, h_ref, wxz_ref,
                whz_ref, wxr_ref, whr_ref, wxh_ref, bz_ref, br_ref, bxh_ref,
                z_ref, xh_ref, ug_ref):
    db = db_ref[...]
    p = (pf_ref[...] + ux_ref[...]) * db
    q = (qf_ref[...] + uh_ref[...]) * db
    dot = lambda a, w: jnp.dot(a, w, preferred_element_type=jnp.float32)
    z = jax.nn.sigmoid(dot(p, wxz_ref[...]) + dot(q, whz_ref[...])
                       + bz_ref[...])
    r = jax.nn.sigmoid(dot(p, wxr_ref[...]) + dot(q, whr_ref[...])
                       + br_ref[...])
    z_ref[...] = z
    xh_ref[...] = dot(p, wxh_ref[...]) + bxh_ref[...]
    ug_ref[...] = r * h_ref[...] * db


def _tc_gates(pf, qf, ux, uh, db, h, wxz, whz, wxr, whr, wxh, bz, br, bxh):
    row = pl.BlockSpec((RB, H), lambda b: (b, 0))
    full = pl.BlockSpec((H, H), lambda b: (0, 0))
    bias = pl.BlockSpec((1, H), lambda b: (0, 0))
    return pl.pallas_call(
        _gates_body,
        grid=(NBLK,),
        in_specs=[row, row, row, row, row, row,
                  full, full, full, full, full, bias, bias, bias],
        out_specs=[row, row, row],
        out_shape=[jax.ShapeDtypeStruct((N, H), jnp.float32),
                   jax.ShapeDtypeStruct((N, H), jnp.float32),
                   jax.ShapeDtypeStruct((N, H), jnp.float32)],
    )(pf, qf, ux, uh, db, h, wxz, whz, wxr, whr, wxh, bz, br, bxh)


def _final_body(sp_ref, ug_ref, db_ref, xh_ref, z_ref, h0_ref, whh_ref,
                bhh_ref, dbn_ref, h_ref, uhn_ref):
    sv = (sp_ref[0] + sp_ref[1] + ug_ref[...]) * db_ref[...]
    hh = jnp.tanh(
        xh_ref[...]
        + jnp.dot(sv, whh_ref[...], preferred_element_type=jnp.float32)
        + bhh_ref[...])
    z = z_ref[...]
    hn = z * h0_ref[...] + (1.0 - z) * hh
    h_ref[...] = hn
    uhn_ref[...] = hn * dbn_ref[...]


def _tc_final(sp, ug, db, xh, z, h0, whh, bhh, dbn):
    row = pl.BlockSpec((RB, H), lambda b: (b, 0))
    full = pl.BlockSpec((H, H), lambda b: (0, 0))
    bias = pl.BlockSpec((1, H), lambda b: (0, 0))
    return pl.pallas_call(
        _final_body,
        grid=(NBLK,),
        in_specs=[pl.BlockSpec((2, RB, H), lambda b: (0, b, 0)),
                  row, row, row, row, bias, full, bias, row],
        out_specs=[row, row],
        out_shape=[jax.ShapeDtypeStruct((N, H), jnp.float32),
                   jax.ShapeDtypeStruct((N, H), jnp.float32)],
    )(sp, ug, db, xh, z, h0, whh, bhh, dbn)


def kernel(xs, eis, W_xz, b_xz, W_hz, b_hz, W_xr, b_xr, W_hr, b_hr,
           W_xh, b_xh, W_hh, b_hh):
    srcs = eis[:, 0, :]
    dsts = eis[:, 1, :]

    deg_parts = _deg_kernel(dsts.reshape(-1))
    dp = deg_parts.reshape(NC, T, NPAD, 16)[:, :, :N, :]
    ux3, db3 = _tc_scale(dp, xs)

    bz = (b_xz + b_hz).reshape(1, H)
    br = (b_xr + b_hr).reshape(1, H)
    bh0 = (b_xh + b_hh).reshape(1, H)
    bxh = b_xh.reshape(1, H)
    bhh = b_hh.reshape(1, H)

    pp = _prop1_kernel(ux3[0], srcs[0], dsts[0])[:, :N, :]
    h, uh = _tc_gates0(pp, ux3[0], db3[0], db3[1], W_xz, W_xh, bz, bh0)
    hs = [h]
    for t in (1, 2):
        pq = _prop2_kernel(ux3[t], uh, srcs[t], dsts[t])[:, :N, :]
        z, xh, ug = _tc_gates(pq[0], pq[1], ux3[t], uh, db3[t], h,
                              W_xz, W_hz, W_xr, W_hr, W_xh, bz, br, bxh)
        sp = _prop1_kernel(ug, srcs[t], dsts[t])[:, :N, :]
        dbn = db3[t + 1] if t + 1 < T else db3[t]
        h, uh = _tc_final(sp, ug, db3[t], xh, z, h[0:1], W_hh, bhh, dbn)
        hs.append(h)
    return jnp.stack(hs)


# speedup vs baseline: 1.4342x; 1.0351x over previous
import functools

import jax
import jax.numpy as jnp
from jax import lax
from jax.experimental import pallas as pl
from jax.experimental.pallas import tpu as pltpu
from jax.experimental.pallas import tpu_sc as plsc

T, N, E, H = 3, 10000, 320000, 128
NC, NS = 2, 16
NPAD = 10240
RPS = NPAD // NS
K = 80
DK = 128
EPD = 327680
RB = 2000
NBLK = N // RB

_vmesh = plsc.VectorSubcoreMesh(core_axis_name="c", subcore_axis_name="s")


def _deg_body(dst_hbm, out_hbm, acc0, acc1, acc2, zbuf, ones, dstv):
    c = lax.axis_index("c")
    s = lax.axis_index("s")
    accs = [acc0, acc1, acc2]

    zero16 = jnp.zeros((16,), jnp.float32)
    one16 = jnp.ones((16,), jnp.float32)

    def fill_z(i, _):
        zbuf[i, :] = zero16
        return 0

    lax.fori_loop(0, 640, fill_z, 0)

    def fill_o(i, _):
        ones[i, :] = one16
        return 0

    lax.fori_loop(0, DK, fill_o, 0)

    for t in range(T):
        pltpu.sync_copy(zbuf, accs[t].at[pl.ds(s * 640, 640), :])
    plsc.subcore_barrier()

    per_sub = EPD // (NC * NS)
    base = c * (EPD // NC) + s * per_sub
    for t in range(T):
        acc = accs[t]
        tbase = t * EPD + base

        def body(g, _):
            pltpu.sync_copy(dst_hbm.at[pl.ds(tbase + g * DK, DK)], dstv)
            pltpu.sync_copy(ones, acc.at[dstv], add=True)
            return 0

        lax.fori_loop(0, per_sub // DK, body, 0)
    plsc.subcore_barrier()

    for t in range(T):
        pltpu.sync_copy(accs[t].at[pl.ds(s * 640, 640), :],
                        out_hbm.at[c * T + t, pl.ds(s * 640, 640), :])


def _zero_acc(acc, zbuf, s):
    zero16 = jnp.zeros((16,), jnp.float32)

    def fill_z(k, _):
        i = k // (H // 16)
        j = k % (H // 16)
        zbuf[i, pl.ds(j * 16, 16)] = zero16
        return 0

    lax.fori_loop(0, 128 * (H // 16), fill_z, 0)
    for r in range(RPS // 128):
        pltpu.sync_copy(zbuf, acc.at[pl.ds(s * RPS + r * 128, 128), :])


def _edge_loop(table_hbm, src_hbm, dst_hbm, acc, srcv, dstv, rows, sem,
               base, n_chunks):
    def body(g, _):
        off = base + g * K
        pltpu.sync_copy(src_hbm.at[pl.ds(off, K)], srcv)
        pltpu.sync_copy(dst_hbm.at[pl.ds(off, K)], dstv)
        pltpu.async_copy(table_hbm.at[srcv], rows, sem).wait()
        pltpu.sync_copy(rows, acc.at[dstv], add=True)
        return 0

    lax.fori_loop(0, n_chunks, body, 0)


def _prop1_body(u_hbm, src_hbm, dst_hbm, out_hbm, acc, zbuf, srcv, dstv, rows,
                sem):
    c = lax.axis_index("c")
    s = lax.axis_index("s")
    _zero_acc(acc, zbuf, s)
    plsc.subcore_barrier()
    per_sub = E // (NC * NS)
    base = c * (E // NC) + s * per_sub
    _edge_loop(u_hbm, src_hbm, dst_hbm, acc, srcv, dstv, rows, sem,
               base, per_sub // K)
    plsc.subcore_barrier()
    pltpu.sync_copy(acc.at[pl.ds(s * RPS, RPS), :],
                    out_hbm.at[c, pl.ds(s * RPS, RPS), :])


def _prop2_body(ua_hbm, ub_hbm, src_hbm, dst_hbm, out_hbm, acc, zbuf, srcv,
                dstv, rows, sem):
    c = lax.axis_index("c")
    s = lax.axis_index("s")
    _zero_acc(acc, zbuf, s)
    plsc.subcore_barrier()
    per_sub = E // NS
    base = s * per_sub

    @pl.when(c == 0)
    def _():
        _edge_loop(ua_hbm, src_hbm, dst_hbm, acc, srcv, dstv, rows, sem,
                   base, per_sub // K)

    @pl.when(c == 1)
    def _():
        _edge_loop(ub_hbm, src_hbm, dst_hbm, acc, srcv, dstv, rows, sem,
                   base, per_sub // K)

    plsc.subcore_barrier()
    pltpu.sync_copy(acc.at[pl.ds(s * RPS, RPS), :],
                    out_hbm.at[c, pl.ds(s * RPS, RPS), :])


_PROP_SCRATCH = [
    pltpu.VMEM_SHARED((NPAD, H), jnp.float32),
    pltpu.VMEM((128, H), jnp.float32),
    pltpu.VMEM((K,), jnp.int32),
    pltpu.VMEM((K,), jnp.int32),
    pltpu.VMEM((K, H), jnp.float32),
    pltpu.SemaphoreType.DMA,
]

_deg_kernel = functools.partial(
    pl.kernel,
    out_type=jax.ShapeDtypeStruct((NC * T, NPAD, 16), jnp.float32),
    mesh=_vmesh,
    scratch_types=[
        pltpu.VMEM_SHARED((NPAD, 16), jnp.float32),
        pltpu.VMEM_SHARED((NPAD, 16), jnp.float32),
        pltpu.VMEM_SHARED((NPAD, 16), jnp.float32),
        pltpu.VMEM((640, 16), jnp.float32),
        pltpu.VMEM((DK, 16), jnp.float32),
        pltpu.VMEM((DK,), jnp.int32),
    ],
)(_deg_body)

_prop1_kernel = functools.partial(
    pl.kernel,
    out_type=jax.ShapeDtypeStruct((NC, NPAD, H), jnp.float32),
    mesh=_vmesh,
    scratch_types=_PROP_SCRATCH,
)(_prop1_body)

_prop2_kernel = functools.partial(
    pl.kernel,
    out_type=jax.ShapeDtypeStruct((NC, NPAD, H), jnp.float32),
    mesh=_vmesh,
    scratch_types=_PROP_SCRATCH,
)(_prop2_body)


def _scale_body(dp_ref, xs_ref, ux_ref, db_ref):
    dp = dp_ref[...]
    deg = dp[0, 0, :, 0:1] + dp[1, 0, :, 0:1] + 1.0
    dinv = lax.rsqrt(deg)
    ux_ref[0] = xs_ref[0] * dinv
    db_ref[0] = jnp.broadcast_to(dinv, (RB, H))


def _tc_scale(dp, xs):
    return pl.pallas_call(
        _scale_body,
        grid=(T, NBLK),
        in_specs=[
            pl.BlockSpec((2, 1, RB, 16), lambda t, b: (0, t, b, 0)),
            pl.BlockSpec((1, RB, H), lambda t, b: (t, b, 0)),
        ],
        out_specs=[
            pl.BlockSpec((1, RB, H), lambda t, b: (t, b, 0)),
            pl.BlockSpec((1, RB, H), lambda t, b: (t, b, 0)),
        ],
        out_shape=[
            jax.ShapeDtypeStruct((T, N, H), jnp.float32),
            jax.ShapeDtypeStruct((T, N, H), jnp.float32),
        ],
    )(dp, xs)


def _gates0_body(pp_ref, ux_ref, db0_ref, db1_ref, wz_ref, wh_ref, bz_ref,
                 bh_ref, h_ref, uh_ref):
    p = (pp_ref[0] + pp_ref[1] + ux_ref[...]) * db0_ref[...]
    z = jax.nn.sigmoid(
        jnp.dot(p, wz_ref[...], preferred_element_type=jnp.float32)
        + bz_ref[...])
    hh = jnp.tanh(
        jnp.dot(p, wh_ref[...], preferred_element_type=jnp.float32)
        + bh_ref[...])
    h1 = (1.0 - z) * hh
    h_ref[...] = h1
    uh_ref[...] = h1 * db1_ref[...]


def _tc_gates0(pp, ux, db0, db1, wz, wh, bz, bh):
    row = pl.BlockSpec((RB, H), lambda b: (b, 0))
    full = pl.BlockSpec((H, H), lambda b: (0, 0))
    bias = pl.BlockSpec((1, H), lambda b: (0, 0))
    return pl.pallas_call(
        _gates0_body,
        grid=(NBLK,),
        in_specs=[pl.BlockSpec((2, RB, H), lambda b: (0, b, 0)),
                  row, row, row, full, full, bias, bias],
        out_specs=[row, row],
        out_shape=[jax.ShapeDtypeStruct((N, H), jnp.float32),
                   jax.ShapeDtypeStruct((N, H), jnp.float32)],
    )(pp, ux, db0, db1, wz, wh, bz, bh)


def _gates_body(pf_ref, qf_ref, ux_ref, uh_ref, db_ref, h_ref, wxz_ref,
                whz_ref, wxr_ref, whr_ref, wxh_ref, bz_ref, br_ref, bxh_ref,
                z_ref, xh_ref, ug_ref):
    db = db_ref[...]
    p = (pf_ref[...] + ux_ref[...]) * db
    q = (qf_ref[...] + uh_ref[...]) * db
    dot = lambda a, w: jnp.dot(a, w, preferred_element_type=jnp.float32)
    z = jax.nn.sigmoid(dot(p, wxz_ref[...]) + dot(q, whz_ref[...])
                       + bz_ref[...])
    r = jax.nn.sigmoid(dot(p, wxr_ref[...]) + dot(q, whr_ref[...])
                       + br_ref[...])
    z_ref[...] = z
    xh_ref[...] = dot(p, wxh_ref[...]) + bxh_ref[...]
    ug_ref[...] = r * h_ref[...] * db


def _tc_gates(pf, qf, ux, uh, db, h, wxz, whz, wxr, whr, wxh, bz, br, bxh):
    row = pl.BlockSpec((RB, H), lambda b: (b, 0))
    full = pl.BlockSpec((H, H), lambda b: (0, 0))
    bias = pl.BlockSpec((1, H), lambda b: (0, 0))
    return pl.pallas_call(
        _gates_body,
        grid=(NBLK,),
        in_specs=[row, row, row, row, row, row,
                  full, full, full, full, full, bias, bias, bias],
        out_specs=[row, row, row],
        out_shape=[jax.ShapeDtypeStruct((N, H), jnp.float32),
                   jax.ShapeDtypeStruct((N, H), jnp.float32),
                   jax.ShapeDtypeStruct((N, H), jnp.float32)],
    )(pf, qf, ux, uh, db, h, wxz, whz, wxr, whr, wxh, bz, br, bxh)


def _final_body(sp_ref, ug_ref, db_ref, xh_ref, z_ref, h0_ref, whh_ref,
                bhh_ref, dbn_ref, h_ref, uhn_ref):
    sv = (sp_ref[0] + sp_ref[1] + ug_ref[...]) * db_ref[...]
    hh = jnp.tanh(
        xh_ref[...]
        + jnp.dot(sv, whh_ref[...], preferred_element_type=jnp.float32)
        + bhh_ref[...])
    z = z_ref[...]
    hn = z * h0_ref[...] + (1.0 - z) * hh
    h_ref[...] = hn
    uhn_ref[...] = hn * dbn_ref[...]


def _tc_final(sp, ug, db, xh, z, h0, whh, bhh, dbn):
    row = pl.BlockSpec((RB, H), lambda b: (b, 0))
    full = pl.BlockSpec((H, H), lambda b: (0, 0))
    bias = pl.BlockSpec((1, H), lambda b: (0, 0))
    return pl.pallas_call(
        _final_body,
        grid=(NBLK,),
        in_specs=[pl.BlockSpec((2, RB, H), lambda b: (0, b, 0)),
                  row, row, row, row, bias, full, bias, row],
        out_specs=[row, row],
        out_shape=[jax.ShapeDtypeStruct((N, H), jnp.float32),
                   jax.ShapeDtypeStruct((N, H), jnp.float32)],
    )(sp, ug, db, xh, z, h0, whh, bhh, dbn)


def kernel(xs, eis, W_xz, b_xz, W_hz, b_hz, W_xr, b_xr, W_hr, b_hr,
           W_xh, b_xh, W_hh, b_hh):
    srcs = eis[:, 0, :]
    dsts = eis[:, 1, :]

    pad_dst = jnp.full((T, EPD - E), NPAD - 1, jnp.int32)
    dsts_pad = jnp.concatenate([dsts, pad_dst], axis=1)
    deg_parts = _deg_kernel(dsts_pad.reshape(-1))
    dp = deg_parts.reshape(NC, T, NPAD, 16)[:, :, :N, :]
    ux3, db3 = _tc_scale(dp, xs)

    bz = (b_xz + b_hz).reshape(1, H)
    br = (b_xr + b_hr).reshape(1, H)
    bh0 = (b_xh + b_hh).reshape(1, H)
    bxh = b_xh.reshape(1, H)
    bhh = b_hh.reshape(1, H)

    pp = _prop1_kernel(ux3[0], srcs[0], dsts[0])[:, :N, :]
    h, uh = _tc_gates0(pp, ux3[0], db3[0], db3[1], W_xz, W_xh, bz, bh0)
    hs = [h]
    for t in (1, 2):
        pq = _prop2_kernel(ux3[t], uh, srcs[t], dsts[t])[:, :N, :]
        z, xh, ug = _tc_gates(pq[0], pq[1], ux3[t], uh, db3[t], h,
                              W_xz, W_hz, W_xr, W_hr, W_xh, bz, br, bxh)
        sp = _prop1_kernel(ug, srcs[t], dsts[t])[:, :N, :]
        dbn = db3[t + 1] if t + 1 < T else db3[t]
        h, uh = _tc_final(sp, ug, db3[t], xh, z, h[0:1], W_hh, bhh, dbn)
        hs.append(h)
    return jnp.stack(hs)
